# baseline (speedup 1.0000x reference)
import functools

import jax
import jax.numpy as jnp
from jax import lax
from jax.experimental import pallas as pl
from jax.experimental.pallas import tpu as pltpu
from jax.experimental.pallas import tpu_sc as plsc

N = 10000
D = 128
R = 3
L = 5

NC = 2
NS = 16
NTILES = NC * NS
CH = 128
NPAD = 10240
RPT = NPAD // NS
ZR = 160

_mesh = plsc.VectorSubcoreMesh(core_axis_name="c", subcore_axis_name="s")


def _pad_edges(edge_index, ept):
    e = edge_index.shape[1]
    pad = NTILES * ept - e
    src = jnp.concatenate([edge_index[0], jnp.zeros((pad,), jnp.int32)])
    pad_dst = N + jnp.arange(pad, dtype=jnp.int32) % (NPAD - N)
    dst = jnp.concatenate([edge_index[1], pad_dst])
    inter = jnp.stack([src.reshape(-1, CH), dst.reshape(-1, CH)], axis=1)
    return inter.reshape(-1)


G = 8


def _make_segsum(ept):
    nchunk = ept // CH
    ngrp = nchunk // G

    @functools.partial(
        pl.kernel,
        out_type=jax.ShapeDtypeStruct((R, NC, NPAD, D), jnp.float32),
        mesh=_mesh,
        scratch_types=[
            pltpu.VMEM((2 * CH,), jnp.int32),
            pltpu.VMEM((CH,), jnp.int32),
            pltpu.VMEM((CH,), jnp.int32),
            pltpu.VMEM((CH, D), jnp.float32),
            pltpu.VMEM((ZR, D), jnp.float32),
            pltpu.VMEM_SHARED((NPAD, D), jnp.float32),
            pltpu.SemaphoreType.DMA,
        ],
    )
    def segsum(h, ed, zrows, out, sdbuf, sidx, didx, rows, zbuf, acc, sem):
        c = lax.axis_index("c")
        s = lax.axis_index("s")
        pltpu.sync_copy(zrows, zbuf)
        ebase = (c * NS + s) * ept
        ep_all = NTILES * ept
        for r in range(R):
            for z in range(RPT // ZR):
                pltpu.sync_copy(zbuf, acc.at[pl.ds(s * RPT + z * ZR, ZR)])
            plsc.subcore_barrier()

            for i in range(CH // 16):
                didx[pl.ds(i * 16, 16)] = N + i * 16 + lax.iota(jnp.int32, 16)
            pltpu.async_copy(rows, acc.at[didx], sem, add=True)

            @pl.loop(0, nchunk)
            def _(j):
                off = 2 * (r * ep_all + ebase + j * CH)
                pltpu.sync_copy(ed.at[pl.ds(off, 2 * CH)], sdbuf)
                for i in range(CH // 16):
                    sidx[pl.ds(i * 16, 16)] = sdbuf[pl.ds(i * 16, 16)]
                pltpu.make_async_copy(rows, acc.at[didx], sem).wait()
                for i in range(CH // 16):
                    didx[pl.ds(i * 16, 16)] = sdbuf[pl.ds(CH + i * 16, 16)]
                pltpu.sync_copy(h.at[sidx], rows)
                pltpu.async_copy(rows, acc.at[didx], sem, add=True)

            pltpu.make_async_copy(rows, acc.at[didx], sem).wait()
            plsc.subcore_barrier()
            pltpu.sync_copy(acc.at[pl.ds(s * RPT, RPT)],
                            out.at[r, c, pl.ds(s * RPT, RPT)])

    return segsum


def _make_counts(ept):
    nchunk = ept // CH

    @functools.partial(
        pl.kernel,
        out_type=jax.ShapeDtypeStruct((R * NC * NPAD,), jnp.float32),
        mesh=_mesh,
        scratch_types=[
            pltpu.VMEM((2 * CH,), jnp.int32),
            pltpu.VMEM((CH,), jnp.int32),
            pltpu.VMEM((CH,), jnp.float32),
            pltpu.VMEM_SHARED((NPAD,), jnp.float32),
            pltpu.SemaphoreType.DMA,
        ],
    )
    def counts(ed, ones_h, zeros_h, out, sdbuf, didx, ones_v, acc, sem):
        c = lax.axis_index("c")
        s = lax.axis_index("s")
        pltpu.sync_copy(ones_h, ones_v)
        ebase = (c * NS + s) * ept
        ep_all = NTILES * ept
        for r in range(R):
            pltpu.sync_copy(zeros_h, acc.at[pl.ds(s * RPT, RPT)])
            plsc.subcore_barrier()

            for i in range(CH // 16):
                didx[pl.ds(i * 16, 16)] = N + i * 16 + lax.iota(jnp.int32, 16)
            pltpu.async_copy(ones_v, acc.at[didx], sem, add=True)

            @pl.loop(0, nchunk)
            def _(j):
                off = 2 * (r * ep_all + ebase + j * CH)
                pltpu.sync_copy(ed.at[pl.ds(off, 2 * CH)], sdbuf)
                pltpu.make_async_copy(ones_v, acc.at[didx], sem).wait()
                for i in range(CH // 16):
                    didx[pl.ds(i * 16, 16)] = sdbuf[pl.ds(CH + i * 16, 16)]
                pltpu.async_copy(ones_v, acc.at[didx], sem, add=True)

            pltpu.make_async_copy(ones_v, acc.at[didx], sem).wait()
            plsc.subcore_barrier()
            pltpu.sync_copy(acc.at[pl.ds(s * RPT, RPT)],
                            out.at[pl.ds((r * NC + c) * NPAD + s * RPT, RPT)])

    return counts


BM = 1000


def _combine_body(h_ref, p_ref, cnt_ref, wsm_ref, wn_ref, bm_ref, o_ref):
    hb = h_ref[...]
    acc = jnp.dot(hb, wsm_ref[...], preferred_element_type=jnp.float32)
    for r in range(R):
        ms = p_ref[r, 0] + p_ref[r, 1]
        cnt = cnt_ref[r, 0] + cnt_ref[r, 1]
        inv = 1.0 / (3.0 * jnp.maximum(cnt, 1.0))
        acc = acc + jnp.dot(ms * inv, wn_ref[r],
                            preferred_element_type=jnp.float32)
    o_ref[...] = acc + bm_ref[...]


def _mid_body(h_ref, p_ref, cnt_ref, wsm_ref, wn_ref, bm_ref, o_ref):
    _combine_body(h_ref, p_ref, cnt_ref, wsm_ref, wn_ref, bm_ref, o_ref)
    t = o_ref[...]
    o_ref[...] = jnp.where(t > 0, t, 0.01 * t)


def _last_body(h_ref, p_ref, cnt_ref, wsm_ref, wn_ref, bm_ref,
               wlin_ref, blin_ref, o_ref):
    _combine_body(h_ref, p_ref, cnt_ref, wsm_ref, wn_ref, bm_ref, o_ref)
    o_ref[...] = jnp.dot(o_ref[...], wlin_ref[...],
                         preferred_element_type=jnp.float32) + blin_ref[...]


def _combine(h, parts, cnt, wsm, wn, bm, wlin=None, blin=None):
    grid = N // BM
    in_specs = [
        pl.BlockSpec((BM, D), lambda i: (i, 0)),
        pl.BlockSpec((R, NC, BM, D), lambda i: (0, 0, i, 0)),
        pl.BlockSpec((R, NC, BM, 1), lambda i: (0, 0, i, 0)),
        pl.BlockSpec((D, D), lambda i: (0, 0)),
        pl.BlockSpec((R, D, D), lambda i: (0, 0, 0)),
        pl.BlockSpec((1, D), lambda i: (0, 0)),
    ]
    args = [h, parts, cnt, wsm, wn, bm.reshape(1, D)]
    if wlin is None:
        body = _mid_body
    else:
        body = _last_body
        in_specs += [pl.BlockSpec((D, D), lambda i: (0, 0)),
                     pl.BlockSpec((1, D), lambda i: (0, 0))]
        args += [wlin, blin.reshape(1, D)]
    return pl.pallas_call(
        body,
        grid=(grid,),
        in_specs=in_specs,
        out_specs=pl.BlockSpec((BM, D), lambda i: (i, 0)),
        out_shape=jax.ShapeDtypeStruct((N, D), jnp.float32),
    )(*args)


def kernel(x, edge_index_r0, edge_index_r1, edge_index_r2,
           Wself, Wneigh, b, Wlin, blin):
    e = edge_index_r0.shape[1]
    ept = -(-e // (NTILES * CH)) * CH

    ed = jnp.concatenate(
        [_pad_edges(ei, ept)
         for ei in (edge_index_r0, edge_index_r1, edge_index_r2)])

    segsum = _make_segsum(ept)
    counts = _make_counts(ept)

    ones_h = jnp.ones((CH,), jnp.float32)
    zeros1 = jnp.zeros((RPT,), jnp.float32)
    zrows = jnp.zeros((ZR, D), jnp.float32)

    cnt = counts(ed, ones_h, zeros1).reshape(R, NC, NPAD, 1)

    wsm = jnp.mean(Wself, axis=1)
    bm = jnp.mean(b, axis=1)

    h = x
    for l in range(L):
        parts = segsum(h, ed, zrows)
        if l < L - 1:
            h = _combine(h, parts, cnt, wsm[l], Wneigh[l], bm[l])
        else:
            h = _combine(h, parts, cnt, wsm[l], Wneigh[l], bm[l], Wlin, blin)
    return h


# speedup vs baseline: 1.0495x; 1.0495x over previous
import functools

import jax
import jax.numpy as jnp
from jax import lax
from jax.experimental import pallas as pl
from jax.experimental.pallas import tpu as pltpu
from jax.experimental.pallas import tpu_sc as plsc

N = 10000
D = 128
R = 3
L = 5

NC = 2
NS = 16
NTILES = NC * NS
CH = 128
NPAD = 10240
RPT = NPAD // NS
ZR = 64

_mesh = plsc.VectorSubcoreMesh(core_axis_name="c", subcore_axis_name="s")


def _pad_edges(edge_index, ept):
    e = edge_index.shape[1]
    pad = NTILES * ept - e
    src = jnp.concatenate([edge_index[0], jnp.zeros((pad,), jnp.int32)])
    pad_dst = N + jnp.arange(pad, dtype=jnp.int32) % (NPAD - N)
    dst = jnp.concatenate([edge_index[1], pad_dst])
    inter = jnp.stack([src.reshape(-1, CH), dst.reshape(-1, CH)], axis=1)
    return inter.reshape(-1)


G = 8


def _make_segsum(ept):
    nchunk = ept // CH
    ngrp = nchunk // G

    @functools.partial(
        pl.kernel,
        out_type=jax.ShapeDtypeStruct((R, NC, NPAD, D), jnp.float32),
        mesh=_mesh,
        scratch_types=[
            pltpu.VMEM((2 * CH,), jnp.int32),
            pltpu.VMEM((CH,), jnp.int32),
            pltpu.VMEM((CH,), jnp.int32),
            pltpu.VMEM((CH, D), jnp.float32),
            pltpu.VMEM((ZR, D), jnp.float32),
            pltpu.VMEM_SHARED((NPAD, D), jnp.float32),
            pltpu.SemaphoreType.DMA,
        ],
    )
    def segsum(h, ed, zrows, out, sdbuf, sidx, didx, rows, zbuf, acc, sem):
        c = lax.axis_index("c")
        s = lax.axis_index("s")
        pltpu.sync_copy(zrows, zbuf)
        ebase = (c * NS + s) * ept
        ep_all = NTILES * ept
        for r in range(R):
            for z in range(RPT // ZR):
                pltpu.sync_copy(zbuf, acc.at[pl.ds(s * RPT + z * ZR, ZR)])
            plsc.subcore_barrier()

            for i in range(CH // 16):
                didx[pl.ds(i * 16, 16)] = N + i * 16 + lax.iota(jnp.int32, 16)
            pltpu.async_copy(rows, acc.at[didx], sem, add=True)

            @pl.loop(0, nchunk)
            def _(j):
                off = 2 * (r * ep_all + ebase + j * CH)
                pltpu.sync_copy(ed.at[pl.ds(off, 2 * CH)], sdbuf)
                for i in range(CH // 16):
                    sidx[pl.ds(i * 16, 16)] = sdbuf[pl.ds(i * 16, 16)]
                pltpu.make_async_copy(rows, acc.at[didx], sem).wait()
                for i in range(CH // 16):
                    didx[pl.ds(i * 16, 16)] = sdbuf[pl.ds(CH + i * 16, 16)]
                pltpu.sync_copy(h.at[sidx], rows)
                pltpu.async_copy(rows, acc.at[didx], sem, add=True)

            pltpu.make_async_copy(rows, acc.at[didx], sem).wait()
            plsc.subcore_barrier()
            pltpu.sync_copy(acc.at[pl.ds(s * RPT, RPT)],
                            out.at[r, c, pl.ds(s * RPT, RPT)])
            plsc.subcore_barrier()

    return segsum


def _make_counts(ept):
    nchunk = ept // CH

    @functools.partial(
        pl.kernel,
        out_type=jax.ShapeDtypeStruct((R * NC * NPAD,), jnp.float32),
        mesh=_mesh,
        scratch_types=[
            pltpu.VMEM((2 * CH,), jnp.int32),
            pltpu.VMEM((CH,), jnp.int32),
            pltpu.VMEM((CH,), jnp.float32),
            pltpu.VMEM_SHARED((NPAD,), jnp.float32),
        ],
    )
    def counts(ed, ones_h, zeros_h, out, sdbuf, didx, ones_v, acc):
        c = lax.axis_index("c")
        s = lax.axis_index("s")
        pltpu.sync_copy(ones_h, ones_v)
        ebase = (c * NS + s) * ept
        ep_all = NTILES * ept
        for r in range(R):
            pltpu.sync_copy(zeros_h, acc.at[pl.ds(s * RPT, RPT)])
            plsc.subcore_barrier()

            @pl.loop(0, nchunk)
            def _(j):
                off = 2 * (r * ep_all + ebase + j * CH)
                pltpu.sync_copy(ed.at[pl.ds(off, 2 * CH)], sdbuf)
                for i in range(CH // 16):
                    didx[pl.ds(i * 16, 16)] = sdbuf[pl.ds(CH + i * 16, 16)]
                pltpu.sync_copy(ones_v, acc.at[didx], add=True)

            plsc.subcore_barrier()
            pltpu.sync_copy(acc.at[pl.ds(s * RPT, RPT)],
                            out.at[pl.ds((r * NC + c) * NPAD + s * RPT, RPT)])
            plsc.subcore_barrier()

    return counts


BM = 1000


def _combine_body(h_ref, p_ref, cnt_ref, wsm_ref, wn_ref, bm_ref, o_ref):
    hb = h_ref[...]
    acc = jnp.dot(hb, wsm_ref[...], preferred_element_type=jnp.float32)
    for r in range(R):
        ms = p_ref[r, 0] + p_ref[r, 1]
        cnt = cnt_ref[r, 0] + cnt_ref[r, 1]
        inv = 1.0 / (3.0 * jnp.maximum(cnt, 1.0))
        acc = acc + jnp.dot(ms * inv, wn_ref[r],
                            preferred_element_type=jnp.float32)
    o_ref[...] = acc + bm_ref[...]


def _mid_body(h_ref, p_ref, cnt_ref, wsm_ref, wn_ref, bm_ref, o_ref):
    _combine_body(h_ref, p_ref, cnt_ref, wsm_ref, wn_ref, bm_ref, o_ref)
    t = o_ref[...]
    o_ref[...] = jnp.where(t > 0, t, 0.01 * t)


def _last_body(h_ref, p_ref, cnt_ref, wsm_ref, wn_ref, bm_ref,
               wlin_ref, blin_ref, o_ref):
    _combine_body(h_ref, p_ref, cnt_ref, wsm_ref, wn_ref, bm_ref, o_ref)
    o_ref[...] = jnp.dot(o_ref[...], wlin_ref[...],
                         preferred_element_type=jnp.float32) + blin_ref[...]


def _combine(h, parts, cnt, wsm, wn, bm, wlin=None, blin=None):
    grid = N // BM
    in_specs = [
        pl.BlockSpec((BM, D), lambda i: (i, 0)),
        pl.BlockSpec((R, NC, BM, D), lambda i: (0, 0, i, 0)),
        pl.BlockSpec((R, NC, BM, 1), lambda i: (0, 0, i, 0)),
        pl.BlockSpec((D, D), lambda i: (0, 0)),
        pl.BlockSpec((R, D, D), lambda i: (0, 0, 0)),
        pl.BlockSpec((1, D), lambda i: (0, 0)),
    ]
    args = [h, parts, cnt, wsm, wn, bm.reshape(1, D)]
    if wlin is None:
        body = _mid_body
    else:
        body = _last_body
        in_specs += [pl.BlockSpec((D, D), lambda i: (0, 0)),
                     pl.BlockSpec((1, D), lambda i: (0, 0))]
        args += [wlin, blin.reshape(1, D)]
    return pl.pallas_call(
        body,
        grid=(grid,),
        in_specs=in_specs,
        out_specs=pl.BlockSpec((BM, D), lambda i: (i, 0)),
        out_shape=jax.ShapeDtypeStruct((N, D), jnp.float32),
    )(*args)


def kernel(x, edge_index_r0, edge_index_r1, edge_index_r2,
           Wself, Wneigh, b, Wlin, blin):
    e = edge_index_r0.shape[1]
    ept = -(-e // (NTILES * CH)) * CH

    ed = jnp.concatenate(
        [_pad_edges(ei, ept)
         for ei in (edge_index_r0, edge_index_r1, edge_index_r2)])

    segsum = _make_segsum(ept)
    counts = _make_counts(ept)

    ones_h = jnp.ones((CH,), jnp.float32)
    zeros1 = jnp.zeros((RPT,), jnp.float32)
    zrows = jnp.zeros((ZR, D), jnp.float32)

    cnt = counts(ed, ones_h, zeros1).reshape(R, NC, NPAD, 1)

    wsm = jnp.mean(Wself, axis=1)
    bm = jnp.mean(b, axis=1)

    h = x
    for l in range(L):
        parts = segsum(h, ed, zrows)
        if l < L - 1:
            h = _combine(h, parts, cnt, wsm[l], Wneigh[l], bm[l])
        else:
            h = _combine(h, parts, cnt, wsm[l], Wneigh[l], bm[l], Wlin, blin)
    return h


# speedup vs baseline: 1.0600x; 1.0100x over previous
import functools

import jax
import jax.numpy as jnp
from jax import lax
from jax.experimental import pallas as pl
from jax.experimental.pallas import tpu as pltpu
from jax.experimental.pallas import tpu_sc as plsc

N = 10000
D = 128
R = 3
L = 5

NC = 2
NS = 16
NTILES = NC * NS
CH = 128
NPAD = 10240
RPT = NPAD // NS
ZR = 64

def _mesh():
    return plsc.VectorSubcoreMesh(core_axis_name="c", subcore_axis_name="s")


def _pad_edges(edge_index, ept):
    e = edge_index.shape[1]
    pad = NTILES * ept - e
    src = jnp.concatenate([edge_index[0], jnp.zeros((pad,), jnp.int32)])
    pad_dst = N + jnp.arange(pad, dtype=jnp.int32) % (NPAD - N)
    dst = jnp.concatenate([edge_index[1], pad_dst])
    inter = jnp.stack([src.reshape(-1, CH), dst.reshape(-1, CH)], axis=1)
    return inter.reshape(-1)


G = 8


def _make_segsum(ept):
    nchunk = ept // CH
    ngrp = nchunk // G

    @functools.partial(
        pl.kernel,
        out_type=jax.ShapeDtypeStruct((R, NC, NPAD, D), jnp.float32),
        mesh=_mesh(),
        scratch_types=[
            pltpu.VMEM((2 * CH,), jnp.int32),
            pltpu.VMEM((CH,), jnp.int32),
            pltpu.VMEM((CH,), jnp.int32),
            pltpu.VMEM((CH, D), jnp.float32),
            pltpu.VMEM((ZR, D), jnp.float32),
            pltpu.VMEM_SHARED((NPAD, D), jnp.float32),
            pltpu.SemaphoreType.DMA,
        ],
    )
    def segsum(h, ed, zrows, out, sdbuf, sidx, didx, rows, zbuf, acc, sem):
        c = lax.axis_index("c")
        s = lax.axis_index("s")
        pltpu.sync_copy(zrows, zbuf)
        ebase = (c * NS + s) * ept
        ep_all = NTILES * ept
        for r in range(R):
            for z in range(RPT // ZR):
                pltpu.sync_copy(zbuf, acc.at[pl.ds(s * RPT + z * ZR, ZR)])
            plsc.subcore_barrier()

            for i in range(CH // 16):
                didx[pl.ds(i * 16, 16)] = N + i * 16 + lax.iota(jnp.int32, 16)
            pltpu.async_copy(rows, acc.at[didx], sem, add=True)

            @pl.loop(0, nchunk)
            def _(j):
                off = 2 * (r * ep_all + ebase + j * CH)
                pltpu.sync_copy(ed.at[pl.ds(off, 2 * CH)], sdbuf)
                for i in range(CH // 16):
                    sidx[pl.ds(i * 16, 16)] = sdbuf[pl.ds(i * 16, 16)]
                pltpu.make_async_copy(rows, acc.at[didx], sem).wait()
                for i in range(CH // 16):
                    didx[pl.ds(i * 16, 16)] = sdbuf[pl.ds(CH + i * 16, 16)]
                pltpu.sync_copy(h.at[sidx], rows)
                pltpu.async_copy(rows, acc.at[didx], sem, add=True)

            pltpu.make_async_copy(rows, acc.at[didx], sem).wait()
            plsc.subcore_barrier()
            pltpu.sync_copy(acc.at[pl.ds(s * RPT, RPT)],
                            out.at[r, c, pl.ds(s * RPT, RPT)])
            plsc.subcore_barrier()

    return segsum


def _make_counts(ept):
    nchunk = ept // CH

    @functools.partial(
        pl.kernel,
        out_type=jax.ShapeDtypeStruct((R * NC * NPAD,), jnp.float32),
        mesh=_mesh(),
        scratch_types=[
            pltpu.VMEM((2 * CH,), jnp.int32),
            pltpu.VMEM((CH,), jnp.int32),
            pltpu.VMEM((CH,), jnp.float32),
            pltpu.VMEM_SHARED((NPAD,), jnp.float32),
        ],
    )
    def counts(ed, ones_h, zeros_h, out, sdbuf, didx, ones_v, acc):
        c = lax.axis_index("c")
        s = lax.axis_index("s")
        pltpu.sync_copy(ones_h, ones_v)
        ebase = (c * NS + s) * ept
        ep_all = NTILES * ept
        for r in range(R):
            pltpu.sync_copy(zeros_h, acc.at[pl.ds(s * RPT, RPT)])
            plsc.subcore_barrier()

            @pl.loop(0, nchunk)
            def _(j):
                off = 2 * (r * ep_all + ebase + j * CH)
                pltpu.sync_copy(ed.at[pl.ds(off, 2 * CH)], sdbuf)
                for i in range(CH // 16):
                    didx[pl.ds(i * 16, 16)] = sdbuf[pl.ds(CH + i * 16, 16)]
                pltpu.sync_copy(ones_v, acc.at[didx], add=True)

            plsc.subcore_barrier()
            pltpu.sync_copy(acc.at[pl.ds(s * RPT, RPT)],
                            out.at[pl.ds((r * NC + c) * NPAD + s * RPT, RPT)])
            plsc.subcore_barrier()

    return counts


BM = 1000


def _combine_body(h_ref, p_ref, cnt_ref, wsm_ref, wn_ref, bm_ref, o_ref):
    hb = h_ref[...]
    acc = jnp.dot(hb, wsm_ref[...], preferred_element_type=jnp.float32)
    for r in range(R):
        ms = p_ref[r, 0] + p_ref[r, 1]
        cnt = cnt_ref[r, 0] + cnt_ref[r, 1]
        inv = 1.0 / (3.0 * jnp.maximum(cnt, 1.0))
        acc = acc + jnp.dot(ms * inv, wn_ref[r],
                            preferred_element_type=jnp.float32)
    o_ref[...] = acc + bm_ref[...]


def _mid_body(h_ref, p_ref, cnt_ref, wsm_ref, wn_ref, bm_ref, o_ref):
    _combine_body(h_ref, p_ref, cnt_ref, wsm_ref, wn_ref, bm_ref, o_ref)
    t = o_ref[...]
    o_ref[...] = jnp.where(t > 0, t, 0.01 * t)


def _last_body(h_ref, p_ref, cnt_ref, wsm_ref, wn_ref, bm_ref,
               wlin_ref, blin_ref, o_ref):
    _combine_body(h_ref, p_ref, cnt_ref, wsm_ref, wn_ref, bm_ref, o_ref)
    o_ref[...] = jnp.dot(o_ref[...], wlin_ref[...],
                         preferred_element_type=jnp.float32) + blin_ref[...]


def _combine(h, parts, cnt, wsm, wn, bm, wlin=None, blin=None):
    grid = N // BM
    in_specs = [
        pl.BlockSpec((BM, D), lambda i: (i, 0)),
        pl.BlockSpec((R, NC, BM, D), lambda i: (0, 0, i, 0)),
        pl.BlockSpec((R, NC, BM, 1), lambda i: (0, 0, i, 0)),
        pl.BlockSpec((D, D), lambda i: (0, 0)),
        pl.BlockSpec((R, D, D), lambda i: (0, 0, 0)),
        pl.BlockSpec((1, D), lambda i: (0, 0)),
    ]
    args = [h, parts, cnt, wsm, wn, bm.reshape(1, D)]
    if wlin is None:
        body = _mid_body
    else:
        body = _last_body
        in_specs += [pl.BlockSpec((D, D), lambda i: (0, 0)),
                     pl.BlockSpec((1, D), lambda i: (0, 0))]
        args += [wlin, blin.reshape(1, D)]
    return pl.pallas_call(
        body,
        grid=(grid,),
        in_specs=in_specs,
        out_specs=pl.BlockSpec((BM, D), lambda i: (i, 0)),
        out_shape=jax.ShapeDtypeStruct((N, D), jnp.float32),
    )(*args)


def kernel(x, edge_index_r0, edge_index_r1, edge_index_r2,
           Wself, Wneigh, b, Wlin, blin):
    e = edge_index_r0.shape[1]
    ept = -(-e // (NTILES * CH)) * CH

    ed = jnp.concatenate(
        [_pad_edges(ei, ept)
         for ei in (edge_index_r0, edge_index_r1, edge_index_r2)])

    segsum = _make_segsum(ept)
    counts = _make_counts(ept)

    ones_h = jnp.ones((CH,), jnp.float32)
    zeros1 = jnp.zeros((RPT,), jnp.float32)
    zrows = jnp.zeros((ZR, D), jnp.float32)

    cnt = counts(ed, ones_h, zeros1).reshape(R, NC, NPAD, 1)

    wsm = jnp.mean(Wself, axis=1)
    bm = jnp.mean(b, axis=1)

    h = x
    for l in range(L):
        parts = segsum(h, ed, zrows)
        if l < L - 1:
            h = _combine(h, parts, cnt, wsm[l], Wneigh[l], bm[l])
        else:
            h = _combine(h, parts, cnt, wsm[l], Wneigh[l], bm[l], Wlin, blin)
    return h
